# Initial kernel scaffold; baseline (speedup 1.0000x reference)
import jax
import jax.numpy as jnp
from jax.experimental import pallas as pl


def kernel(start_node, end_node, path, node_table, path_table, W, b):
    raise NotImplementedError("write your pallas kernel here")


# speedup vs baseline: 1.7916x; 1.7916x over previous
import functools

import jax
import jax.numpy as jnp
from jax import lax
from jax.experimental import pallas as pl
from jax.experimental.pallas import tpu as pltpu
from jax.experimental.pallas import tpu_sc as plsc

_NODE_SIZE = 100000
_PATH_SIZE = 100
_D = 128
_B = 16384
_L = 16
_NC, _NS = 2, 16
_NW = _NC * _NS
_BPW = _B // _NW
_C = 128
_NCHUNK = _BPW // _C


def _lane_perm(x, idx):
    dn = lax.GatherDimensionNumbers(offset_dims=(), collapsed_slice_dims=(0,),
                                    start_index_map=(0,))
    return lax.gather(x, idx[:, None], dn, slice_sizes=(1,),
                      mode=lax.GatherScatterMode.PROMISE_IN_BOUNDS)


def _hin2vec_body(start_hbm, end_hbm, path_hbm, node_hbm, ptab_hbm, w_hbm,
                  b_hbm, out_hbm, sidx, eidx, pidx, srows, erows, prows,
                  wv, bv, ostage, sem):
    wid = lax.axis_index("s") * _NC + lax.axis_index("c")
    base = wid * _BPW

    pltpu.sync_copy(w_hbm, wv)
    pltpu.sync_copy(b_hbm, bv)
    wregs = [wv[pl.ds(16 * j, _L)] for j in range(_D // _L)]
    bvec = bv[...]
    lane = lax.iota(jnp.int32, _L)
    perm = [(lane + sh) & (_L - 1) for sh in (8, 4, 2, 1)]

    for c in range(_NCHUNK):
        off = base + c * _C
        pltpu.sync_copy(start_hbm.at[pl.ds(off, _C)], sidx)
        pltpu.sync_copy(end_hbm.at[pl.ds(off, _C)], eidx)
        pltpu.sync_copy(path_hbm.at[pl.ds(off, _C)], pidx)
        cp_s = pltpu.async_copy(node_hbm.at[sidx], srows, sem)
        cp_e = pltpu.async_copy(node_hbm.at[eidx], erows, sem)
        cp_p = pltpu.async_copy(ptab_hbm.at[pidx], prows, sem)
        cp_s.wait()
        cp_e.wait()
        cp_p.wait()

        def group_body(g, _):
            def row_body(i, resvec):
                row = g * _L + i
                acc = jnp.zeros((_L,), jnp.float32)
                for j in range(_D // _L):
                    s = srows[row, pl.ds(16 * j, _L)]
                    e = erows[row, pl.ds(16 * j, _L)]
                    p = prows[row, pl.ds(16 * j, _L)]
                    sig = 1.0 / (1.0 + jnp.exp(-p))
                    acc = acc + (s * e) * (wregs[j] * sig)
                for pidx_v in perm:
                    acc = acc + _lane_perm(acc, pidx_v)
                return jnp.where(lane == i, acc, resvec)

            resvec = lax.fori_loop(0, _L, row_body,
                                   jnp.zeros((_L,), jnp.float32))
            z = resvec + bvec
            outv = 1.0 / (1.0 + jnp.exp(-z))
            ostage[pl.ds(c * _C + g * _L, _L)] = outv
            return 0

        lax.fori_loop(0, _C // _L, group_body, 0)

    pltpu.sync_copy(ostage, out_hbm.at[pl.ds(base, _BPW)])


@jax.jit
def _hin2vec_sc(start_i, end_i, path_i, node_table, path_table, w_flat,
                b_vec):
    mesh = plsc.VectorSubcoreMesh(core_axis_name="c", subcore_axis_name="s")
    f = functools.partial(
        pl.kernel,
        mesh=mesh,
        out_type=jax.ShapeDtypeStruct((_B,), jnp.float32),
        scratch_types=[
            pltpu.VMEM((_C,), jnp.int32),
            pltpu.VMEM((_C,), jnp.int32),
            pltpu.VMEM((_C,), jnp.int32),
            pltpu.VMEM((_C, _D), jnp.float32),
            pltpu.VMEM((_C, _D), jnp.float32),
            pltpu.VMEM((_C, _D), jnp.float32),
            pltpu.VMEM((_D,), jnp.float32),
            pltpu.VMEM((_L,), jnp.float32),
            pltpu.VMEM((_BPW,), jnp.float32),
            pltpu.SemaphoreType.DMA,
        ],
    )(_hin2vec_body)
    return f(start_i, end_i, path_i, node_table, path_table, w_flat, b_vec)


def kernel(start_node, end_node, path, node_table, path_table, W, b):
    start_i = start_node.astype(jnp.int32)
    end_i = end_node.astype(jnp.int32)
    path_i = path.astype(jnp.int32)
    w_flat = W.reshape(_D)
    b_vec = jnp.broadcast_to(b.reshape(()), (_L,)).astype(jnp.float32)
    out = _hin2vec_sc(start_i, end_i, path_i, node_table, path_table,
                      w_flat, b_vec)
    return out.reshape(_B, 1)


# speedup vs baseline: 2.0177x; 1.1262x over previous
import functools

import jax
import jax.numpy as jnp
from jax import lax
from jax.experimental import pallas as pl
from jax.experimental.pallas import tpu as pltpu
from jax.experimental.pallas import tpu_sc as plsc

_NODE_SIZE = 100000
_PATH_SIZE = 100
_D = 128
_B = 16384
_L = 16
_NC, _NS = 2, 16
_NW = _NC * _NS
_BPW = _B // _NW
_C = 128
_NCHUNK = _BPW // _C


def _lane_perm(x, idx):
    dn = lax.GatherDimensionNumbers(offset_dims=(), collapsed_slice_dims=(0,),
                                    start_index_map=(0,))
    return lax.gather(x, idx[:, None], dn, slice_sizes=(1,),
                      mode=lax.GatherScatterMode.PROMISE_IN_BOUNDS)


def _sigmoid(x):
    return 1.0 / (1.0 + jnp.exp(-x))


def _hin2vec_body(start_hbm, end_hbm, path_hbm, node_hbm, ptab_hbm, w_hbm,
                  b_hbm, out_hbm, sidx, eidx, pidx, srows, erows, prows,
                  wv, bv, ostage, sem0, sem1):
    cid = lax.axis_index("c")
    sid = lax.axis_index("s")
    wid = sid * _NC + cid
    base = wid * _BPW

    pltpu.sync_copy(w_hbm, wv)
    pltpu.sync_copy(b_hbm, bv)
    pltpu.sync_copy(start_hbm.at[pl.ds(base, _BPW)], sidx)
    pltpu.sync_copy(end_hbm.at[pl.ds(base, _BPW)], eidx)
    pltpu.sync_copy(path_hbm.at[pl.ds(base, _BPW)], pidx)

    wregs = [wv[pl.ds(16 * j, _L)] for j in range(_D // _L)]
    bvec = bv[...]
    lane = lax.iota(jnp.int32, _L)
    perm = [(lane + sh) & (_L - 1) for sh in (8, 4, 2, 1)]

    sems = [sem0, sem1]

    def fire(c):
        half = c % 2
        sl = pl.ds(c * _C, _C)
        sem = sems[half]
        hs = pl.ds(half * _C, _C)
        return (
            pltpu.async_copy(node_hbm.at[sidx.at[sl]], srows.at[hs], sem),
            pltpu.async_copy(node_hbm.at[eidx.at[sl]], erows.at[hs], sem),
            pltpu.async_copy(ptab_hbm.at[pidx.at[sl]], prows.at[hs], sem),
        )

    pending = fire(0)
    for c in range(_NCHUNK):
        nxt = fire(c + 1) if c + 1 < _NCHUNK else None
        for cp in pending:
            cp.wait()
        pending = nxt
        rbase = (c % 2) * _C

        def group_body(g, _):
            def row_body(i, resvec):
                row = rbase + g * _L + i
                acc = jnp.zeros((_L,), jnp.float32)
                for j in range(_D // _L):
                    s = srows[row, pl.ds(16 * j, _L)]
                    e = erows[row, pl.ds(16 * j, _L)]
                    p = prows[row, pl.ds(16 * j, _L)]
                    acc = acc + (s * e) * (wregs[j] * _sigmoid(p))
                for pidx_v in perm:
                    acc = acc + _lane_perm(acc, pidx_v)
                return jnp.where(lane == i, acc, resvec)

            resvec = lax.fori_loop(0, _L, row_body,
                                   jnp.zeros((_L,), jnp.float32))
            outv = _sigmoid(resvec + bvec)
            ostage[pl.ds(c * _C + g * _L, _L)] = outv
            return 0

        lax.fori_loop(0, _C // _L, group_body, 0)

    pltpu.sync_copy(ostage, out_hbm.at[pl.ds(base, _BPW)])


@jax.jit
def _hin2vec_sc(start_i, end_i, path_i, node_table, path_table, w_flat,
                b_vec):
    mesh = plsc.VectorSubcoreMesh(core_axis_name="c", subcore_axis_name="s")
    f = functools.partial(
        pl.kernel,
        mesh=mesh,
        out_type=jax.ShapeDtypeStruct((_B,), jnp.float32),
        scratch_types=[
            pltpu.VMEM((_BPW,), jnp.int32),
            pltpu.VMEM((_BPW,), jnp.int32),
            pltpu.VMEM((_BPW,), jnp.int32),
            pltpu.VMEM((2 * _C, _D), jnp.float32),
            pltpu.VMEM((2 * _C, _D), jnp.float32),
            pltpu.VMEM((2 * _C, _D), jnp.float32),
            pltpu.VMEM((_D,), jnp.float32),
            pltpu.VMEM((_L,), jnp.float32),
            pltpu.VMEM((_BPW,), jnp.float32),
            pltpu.SemaphoreType.DMA,
            pltpu.SemaphoreType.DMA,
        ],
    )(_hin2vec_body)
    return f(start_i, end_i, path_i, node_table, path_table, w_flat, b_vec)


def kernel(start_node, end_node, path, node_table, path_table, W, b):
    start_i = start_node.astype(jnp.int32)
    end_i = end_node.astype(jnp.int32)
    path_i = path.astype(jnp.int32)
    w_flat = W.reshape(_D)
    b_vec = jnp.broadcast_to(b.reshape(()), (_L,)).astype(jnp.float32)
    out = _hin2vec_sc(start_i, end_i, path_i, node_table, path_table,
                      w_flat, b_vec)
    return out.reshape(_B, 1)


# speedup vs baseline: 2.3830x; 1.1811x over previous
import functools

import jax
import jax.numpy as jnp
from jax import lax
from jax.experimental import pallas as pl
from jax.experimental.pallas import tpu as pltpu
from jax.experimental.pallas import tpu_sc as plsc

_NODE_SIZE = 100000
_PATH_SIZE = 100
_D = 128
_B = 16384
_L = 16
_NC, _NS = 2, 16
_NW = _NC * _NS
_BPW = _B // _NW
_C = 128
_NCHUNK = _BPW // _C
_PROWS = 8
_PPAD = _PROWS * _NS


def _lane_perm(x, idx):
    dn = lax.GatherDimensionNumbers(offset_dims=(), collapsed_slice_dims=(0,),
                                    start_index_map=(0,))
    return lax.gather(x, idx[:, None], dn, slice_sizes=(1,),
                      mode=lax.GatherScatterMode.PROMISE_IN_BOUNDS)


def _sigmoid(x):
    return 1.0 / (1.0 + jnp.exp(-x))


def _hin2vec_body(start_hbm, end_hbm, path_hbm, node_hbm, ptab_hbm, w_hbm,
                  b_hbm, out_hbm, sidx, eidx, pidx, srows, erows, pwtab,
                  wv, bv, ptmp, ostage, pw_shared, sem0, sem1):
    cid = lax.axis_index("c")
    sid = lax.axis_index("s")
    wid = sid * _NC + cid
    base = wid * _BPW

    pltpu.sync_copy(w_hbm, wv)
    pltpu.sync_copy(b_hbm, bv)
    pltpu.sync_copy(start_hbm.at[pl.ds(base, _BPW)], sidx)
    pltpu.sync_copy(end_hbm.at[pl.ds(base, _BPW)], eidx)
    pltpu.sync_copy(path_hbm.at[pl.ds(base, _BPW)], pidx.at[pl.ds(0, _BPW)])

    wregs = [wv[pl.ds(16 * j, _L)] for j in range(_D // _L)]
    bvec = bv[...]
    lane = lax.iota(jnp.int32, _L)
    perm = [(lane + sh) & (_L - 1) for sh in (8, 4, 2, 1)]

    prow0 = sid * _PROWS
    pltpu.sync_copy(ptab_hbm.at[pl.ds(prow0, _PROWS)], ptmp)
    for r in range(_PROWS):
        for j in range(_D // _L):
            v = ptmp[r, pl.ds(16 * j, _L)]
            ptmp[r, pl.ds(16 * j, _L)] = _sigmoid(v) * wregs[j]
    pltpu.sync_copy(ptmp, pw_shared.at[pl.ds(prow0, _PROWS)])
    plsc.subcore_barrier()
    pltpu.sync_copy(pw_shared, pwtab)

    sems = [sem0, sem1]

    def fire(c):
        half = c % 2
        sl = pl.ds(c * _C, _C)
        sem = sems[half]
        hs = pl.ds(half * _C, _C)
        return (
            pltpu.async_copy(node_hbm.at[sidx.at[sl]], srows.at[hs], sem),
            pltpu.async_copy(node_hbm.at[eidx.at[sl]], erows.at[hs], sem),
        )

    pending = fire(0)
    for c in range(_NCHUNK):
        nxt = fire(c + 1) if c + 1 < _NCHUNK else None
        for cp in pending:
            cp.wait()
        pending = nxt
        rbase = (c % 2) * _C

        def group_body(g, _):
            def row_body(i, resvec):
                row = rbase + g * _L + i
                pvec = pidx[pl.ds(c * _C + g * _L + i, _L)]
                prow = pvec[0]
                acc = jnp.zeros((_L,), jnp.float32)
                for j in range(_D // _L):
                    s = srows[row, pl.ds(16 * j, _L)]
                    e = erows[row, pl.ds(16 * j, _L)]
                    pw = pwtab[prow, pl.ds(16 * j, _L)]
                    acc = acc + (s * e) * pw
                for pidx_v in perm:
                    acc = acc + _lane_perm(acc, pidx_v)
                return jnp.where(lane == i, acc, resvec)

            resvec = lax.fori_loop(0, _L, row_body,
                                   jnp.zeros((_L,), jnp.float32))
            outv = _sigmoid(resvec + bvec)
            ostage[pl.ds(c * _C + g * _L, _L)] = outv
            return 0

        lax.fori_loop(0, _C // _L, group_body, 0)

    pltpu.sync_copy(ostage, out_hbm.at[pl.ds(base, _BPW)])


@jax.jit
def _hin2vec_sc(start_i, end_i, path_i, node_table, ptab_pad, w_flat, b_vec):
    mesh = plsc.VectorSubcoreMesh(core_axis_name="c", subcore_axis_name="s")
    f = functools.partial(
        pl.kernel,
        mesh=mesh,
        out_type=jax.ShapeDtypeStruct((_B,), jnp.float32),
        scratch_types=[
            pltpu.VMEM((_BPW,), jnp.int32),
            pltpu.VMEM((_BPW,), jnp.int32),
            pltpu.VMEM((_BPW + _L,), jnp.int32),
            pltpu.VMEM((2 * _C, _D), jnp.float32),
            pltpu.VMEM((2 * _C, _D), jnp.float32),
            pltpu.VMEM((_PPAD, _D), jnp.float32),
            pltpu.VMEM((_D,), jnp.float32),
            pltpu.VMEM((_L,), jnp.float32),
            pltpu.VMEM((_PROWS, _D), jnp.float32),
            pltpu.VMEM((_BPW,), jnp.float32),
            pltpu.VMEM_SHARED((_PPAD, _D), jnp.float32),
            pltpu.SemaphoreType.DMA,
            pltpu.SemaphoreType.DMA,
        ],
    )(_hin2vec_body)
    return f(start_i, end_i, path_i, node_table, ptab_pad, w_flat, b_vec)


def kernel(start_node, end_node, path, node_table, path_table, W, b):
    start_i = start_node.astype(jnp.int32)
    end_i = end_node.astype(jnp.int32)
    path_i = path.astype(jnp.int32)
    ptab_pad = jnp.pad(path_table, ((0, _PPAD - _PATH_SIZE), (0, 0)))
    w_flat = W.reshape(_D)
    b_vec = jnp.broadcast_to(b.reshape(()), (_L,)).astype(jnp.float32)
    out = _hin2vec_sc(start_i, end_i, path_i, node_table, ptab_pad,
                      w_flat, b_vec)
    return out.reshape(_B, 1)
